# baseline (speedup 1.0000x reference)
import jax
import jax.numpy as jnp
from jax.experimental import pallas as pl
from jax.experimental.pallas import tpu as pltpu

CH = 200
NBUF = 4
NOBUF = 2


def _gcn_manual(x_ref, adj_ref, w_ref, b_ref, out_ref,
                h_ref, buf_ref, obuf_ref, in_sem, out_sem):
    n = x_ref.shape[0]
    nchunk = n // CH

    def in_copy(j, slot):
        return pltpu.make_async_copy(
            adj_ref.at[pl.ds(j * CH, CH), :], buf_ref.at[slot], in_sem.at[slot])

    def out_copy(j, oslot):
        return pltpu.make_async_copy(
            obuf_ref.at[oslot], out_ref.at[pl.ds(j * CH, CH), :],
            out_sem.at[oslot])

    for j in range(NBUF):
        in_copy(j, j).start()

    h_ref[...] = jnp.dot(x_ref[...], w_ref[...],
                         preferred_element_type=jnp.float32)

    def body(j, carry):
        slot = jax.lax.rem(j, NBUF)
        oslot = jax.lax.rem(j, NOBUF)
        in_copy(j, slot).wait()

        @pl.when(j >= NOBUF)
        def _():
            out_copy(j - NOBUF, oslot).wait()

        obuf_ref[oslot] = jnp.dot(buf_ref[slot], h_ref[...],
                                  preferred_element_type=jnp.float32) + b_ref[...]
        out_copy(j, oslot).start()

        @pl.when(j + NBUF < nchunk)
        def _():
            in_copy(j + NBUF, slot).start()

        return carry

    jax.lax.fori_loop(0, nchunk, body, 0)

    for k in range(NOBUF):
        jd = nchunk - NOBUF + k
        out_copy(jd, jd % NOBUF).wait()


@jax.jit
def kernel(x, adj, W, b):
    n, in_dim = x.shape
    out_dim = W.shape[1]
    return pl.pallas_call(
        _gcn_manual,
        grid=(),
        in_specs=[
            pl.BlockSpec(memory_space=pltpu.VMEM),
            pl.BlockSpec(memory_space=pl.ANY),
            pl.BlockSpec(memory_space=pltpu.VMEM),
            pl.BlockSpec(memory_space=pltpu.VMEM),
        ],
        out_specs=pl.BlockSpec(memory_space=pl.ANY),
        out_shape=jax.ShapeDtypeStruct((n, out_dim), jnp.float32),
        scratch_shapes=[
            pltpu.VMEM((n, out_dim), jnp.float32),
            pltpu.VMEM((NBUF, CH, n), jnp.float32),
            pltpu.VMEM((NOBUF, CH, out_dim), jnp.float32),
            pltpu.SemaphoreType.DMA((NBUF,)),
            pltpu.SemaphoreType.DMA((NOBUF,)),
        ],
    )(x, adj, W, b.reshape(1, out_dim))


# speedup vs baseline: 1.0040x; 1.0040x over previous
import jax
import jax.numpy as jnp
from jax.experimental import pallas as pl
from jax.experimental.pallas import tpu as pltpu

CH = 400
NBUF = 2
NOBUF = 2


def _gcn_manual(x_ref, adj_ref, w_ref, b_ref, out_ref,
                h_ref, buf_ref, obuf_ref, in_sem, out_sem):
    n = x_ref.shape[0]
    nchunk = n // CH

    def in_copy(j, slot):
        return pltpu.make_async_copy(
            adj_ref.at[pl.ds(j * CH, CH), :], buf_ref.at[slot], in_sem.at[slot])

    def out_copy(j, oslot):
        return pltpu.make_async_copy(
            obuf_ref.at[oslot], out_ref.at[pl.ds(j * CH, CH), :],
            out_sem.at[oslot])

    for j in range(NBUF):
        in_copy(j, j).start()

    h_ref[...] = jnp.dot(x_ref[...], w_ref[...],
                         preferred_element_type=jnp.float32)

    def body(j, carry):
        slot = jax.lax.rem(j, NBUF)
        oslot = jax.lax.rem(j, NOBUF)
        in_copy(j, slot).wait()

        @pl.when(j >= NOBUF)
        def _():
            out_copy(j - NOBUF, oslot).wait()

        obuf_ref[oslot] = jnp.dot(buf_ref[slot], h_ref[...],
                                  preferred_element_type=jnp.float32) + b_ref[...]
        out_copy(j, oslot).start()

        @pl.when(j + NBUF < nchunk)
        def _():
            in_copy(j + NBUF, slot).start()

        return carry

    jax.lax.fori_loop(0, nchunk, body, 0)

    for k in range(NOBUF):
        jd = nchunk - NOBUF + k
        out_copy(jd, jd % NOBUF).wait()


@jax.jit
def kernel(x, adj, W, b):
    n, in_dim = x.shape
    out_dim = W.shape[1]
    return pl.pallas_call(
        _gcn_manual,
        grid=(),
        in_specs=[
            pl.BlockSpec(memory_space=pltpu.VMEM),
            pl.BlockSpec(memory_space=pl.ANY),
            pl.BlockSpec(memory_space=pltpu.VMEM),
            pl.BlockSpec(memory_space=pltpu.VMEM),
        ],
        out_specs=pl.BlockSpec(memory_space=pl.ANY),
        out_shape=jax.ShapeDtypeStruct((n, out_dim), jnp.float32),
        scratch_shapes=[
            pltpu.VMEM((n, out_dim), jnp.float32),
            pltpu.VMEM((NBUF, CH, n), jnp.float32),
            pltpu.VMEM((NOBUF, CH, out_dim), jnp.float32),
            pltpu.SemaphoreType.DMA((NBUF,)),
            pltpu.SemaphoreType.DMA((NOBUF,)),
        ],
    )(x, adj, W, b.reshape(1, out_dim))


# speedup vs baseline: 1.0116x; 1.0076x over previous
import functools

import jax
import jax.numpy as jnp
from jax.experimental import pallas as pl
from jax.experimental.pallas import tpu as pltpu

N = 10000
BM = 400


def _gcn_kernel(x_ref, adj_ref, w_ref, b_ref, out_ref, h_ref):
    @pl.when(pl.program_id(0) == 0)
    def _():
        h_ref[...] = jnp.dot(x_ref[...], w_ref[...],
                             preferred_element_type=jnp.float32)

    out_ref[...] = jnp.dot(adj_ref[...], h_ref[...],
                           preferred_element_type=jnp.float32) + b_ref[...]


@jax.jit
def kernel(x, adj, W, b):
    n, in_dim = x.shape
    out_dim = W.shape[1]
    grid = (pl.cdiv(n, BM),)
    return pl.pallas_call(
        _gcn_kernel,
        grid=grid,
        in_specs=[
            pl.BlockSpec((n, in_dim), lambda i: (0, 0)),
            pl.BlockSpec((BM, n), lambda i: (i, 0)),
            pl.BlockSpec((in_dim, out_dim), lambda i: (0, 0)),
            pl.BlockSpec((1, out_dim), lambda i: (0, 0)),
        ],
        out_specs=pl.BlockSpec((BM, out_dim), lambda i: (i, 0)),
        out_shape=jax.ShapeDtypeStruct((n, out_dim), jnp.float32),
        scratch_shapes=[pltpu.VMEM((n, out_dim), jnp.float32)],
        compiler_params=pltpu.CompilerParams(
            dimension_semantics=("arbitrary",),
            vmem_limit_bytes=64 * 1024 * 1024,
        ),
    )(x, adj, W, b.reshape(1, out_dim))
